# baseline (speedup 1.0000x reference)
import functools
import jax
import jax.numpy as jnp
from jax import lax
from jax.experimental import pallas as pl
from jax.experimental.pallas import tpu as pltpu
from jax.experimental.pallas import tpu_sc as plsc

P = 4096
K = 16
C = 13
CP = 16
RADIUS2 = 0.01
RB = 256
NTILES = 32
PPT = 256
CHUNK = 128


def _lane_cumsum(x):
    n = x.shape[-1]
    lane = jax.lax.broadcasted_iota(jnp.int32, x.shape, len(x.shape) - 1)
    s = 1
    while s < n:
        shifted = pltpu.roll(x, s, axis=len(x.shape) - 1)
        x = x + jnp.where(lane >= s, shifted, 0.0)
        s *= 2
    return x


def _select_body(pts_r_ref, ptsT_ref, log_r_ref, idx_ref, scale_ref, offs_ref):
    n = pl.program_id(0)
    i = pl.program_id(1)

    x = pts_r_ref[0]
    xT = ptsT_ref[0]
    d2 = jnp.zeros((RB, P), jnp.float32)
    for d in range(3):
        t = x[:, d:d + 1] - xT[d:d + 1, :]
        d2 = d2 + t * t
    within = (d2 < RADIUS2).astype(jnp.float32)

    rank = _lane_cumsum(within)
    sel = within * (rank <= K).astype(jnp.float32)
    rank_m = sel * rank

    total = jnp.sum(within, axis=1, keepdims=True)
    length = jnp.minimum(total, float(K))
    scale = 1.0 / (C * length)

    j_abs = (jax.lax.broadcasted_iota(jnp.int32, (RB, P), 1).astype(jnp.float32)
             + (n * P).astype(jnp.float32))
    v = sel * j_abs
    g_abs = (jax.lax.broadcasted_iota(jnp.int32, (RB, 1), 0).astype(jnp.float32)
             + (n * P + i * RB).astype(jnp.float32))

    for k in range(1, K + 1):
        col = jnp.sum(jnp.where(rank_m == float(k), v, 0.0), axis=1,
                      keepdims=True)
        col = jnp.where(float(k) <= length, col, g_abs)
        idx_ref[0, :, k - 1:k] = col.astype(jnp.int32)

    scale_ref[0] = jnp.broadcast_to(scale, (RB, CP))

    lg = log_r_ref[0]
    m = jnp.sum(jnp.abs(lg), axis=1, keepdims=True)
    part = jnp.sum((K - length) * m * scale).reshape(1, 1)

    first = jnp.logical_and(n == 0, i == 0)

    @pl.when(first)
    def _():
        offs_ref[...] = part

    @pl.when(jnp.logical_not(first))
    def _():
        offs_ref[...] = offs_ref[...] + part


def _tc_select(points, logits):
    N = points.shape[0]
    ptsT = points.transpose(0, 2, 1)
    return pl.pallas_call(
        _select_body,
        grid=(N, P // RB),
        in_specs=[
            pl.BlockSpec((1, RB, 3), lambda n, i: (n, i, 0)),
            pl.BlockSpec((1, 3, P), lambda n, i: (n, 0, 0)),
            pl.BlockSpec((1, RB, C), lambda n, i: (n, i, 0)),
        ],
        out_specs=[
            pl.BlockSpec((1, RB, K), lambda n, i: (n, i, 0)),
            pl.BlockSpec((1, RB, CP), lambda n, i: (n, i, 0)),
            pl.BlockSpec((1, 1), lambda n, i: (0, 0)),
        ],
        out_shape=[
            jax.ShapeDtypeStruct((N, P, K), jnp.int32),
            jax.ShapeDtypeStruct((N, P, CP), jnp.float32),
            jax.ShapeDtypeStruct((1, 1), jnp.float32),
        ],
    )(points, ptsT, logits)


def _sc_gather_loss(logits_pad, idx_flat, scale2d):
    mesh = plsc.VectorSubcoreMesh(core_axis_name="c", subcore_axis_name="s")

    @functools.partial(
        pl.kernel,
        mesh=mesh,
        compiler_params=pltpu.CompilerParams(use_tc_tiling_on_sc=False),
        out_type=jax.ShapeDtypeStruct((NTILES, CP), jnp.float32),
        scratch_types=[
            pltpu.VMEM((PPT, CP), jnp.float32),
            pltpu.VMEM((PPT, CP), jnp.float32),
            pltpu.VMEM((PPT * K,), jnp.int32),
            pltpu.VMEM((PPT * K, CP), jnp.float32),
            pltpu.VMEM((CP,), jnp.float32),
            pltpu.SemaphoreType.DMA,
        ],
    )
    def sc_kernel(log_hbm, idx_hbm, scale_hbm, out_hbm,
                  own_v, scale_v, idx_v, rows_v, acc_v, sem):
        wid = lax.axis_index("s") * 2 + lax.axis_index("c")
        base_pt = wid * PPT

        pltpu.sync_copy(log_hbm.at[pl.ds(base_pt, PPT)], own_v)
        pltpu.sync_copy(scale_hbm.at[pl.ds(base_pt, PPT)], scale_v)
        pltpu.sync_copy(idx_hbm.at[pl.ds(base_pt * K, PPT * K)], idx_v)

        def gather_chunk(c, carry):
            pltpu.async_copy(
                log_hbm.at[idx_v.at[pl.ds(c * CHUNK, CHUNK)]],
                rows_v.at[pl.ds(c * CHUNK, CHUNK)],
                sem,
            ).wait()
            return carry

        lax.fori_loop(0, (PPT * K) // CHUNK, gather_chunk, 0)

        def point_body(p, acc):
            own = own_v[p, :]
            sv = scale_v[p, :]
            for k in range(K):
                nb = rows_v[p * K + k, :]
                acc = acc + sv * jnp.abs(nb - own)
            return acc

        acc = lax.fori_loop(0, PPT, point_body,
                            jnp.zeros((CP,), jnp.float32))
        acc_v[...] = acc
        pltpu.sync_copy(acc_v, out_hbm.at[wid])

    return sc_kernel(logits_pad, idx_flat, scale2d)


def kernel(points, logits):
    N = points.shape[0]
    idx, scale, offs = _tc_select(points, logits)
    logits_pad = jnp.pad(logits, ((0, 0), (0, 0), (0, CP - C)))
    logits_pad = logits_pad.reshape(N * P, CP)
    idx_flat = idx.reshape(N * P * K)
    scale2d = scale.reshape(N * P, CP)
    partials = _sc_gather_loss(logits_pad, idx_flat, scale2d)
    return (jnp.sum(partials) + offs[0, 0]) / (N * P)


# speedup vs baseline: 2.2514x; 2.2514x over previous
import functools
import jax
import jax.numpy as jnp
from jax import lax
from jax.experimental import pallas as pl
from jax.experimental.pallas import tpu as pltpu
from jax.experimental.pallas import tpu_sc as plsc

P = 4096
K = 16
C = 13
CP = 16
RADIUS2 = 0.01
RB = 256
W = 16
NQ = P // W
NTILES = 32
PPT = 256
HALF = 128
CHUNK = 128


def _pack_body(pts_r_ref, ptsT_ref, log_r_ref, packed_ref, scale_ref, offs_ref):
    n = pl.program_id(0)
    i = pl.program_id(1)

    x = pts_r_ref[0]
    words = jnp.zeros((RB, NQ), jnp.int32)
    cnt = jnp.zeros((RB, NQ), jnp.float32)
    for b in range(W):
        xb = ptsT_ref[0, b]
        d2 = jnp.zeros((RB, NQ), jnp.float32)
        for d in range(3):
            t = x[:, d:d + 1] - xb[d:d + 1, :]
            d2 = d2 + t * t
        wb = d2 < RADIUS2
        words = words + jnp.where(wb, jnp.int32(1 << b), jnp.int32(0))
        cnt = cnt + wb.astype(jnp.float32)
    packed_ref[0] = words

    total = jnp.sum(cnt, axis=1, keepdims=True)
    length = jnp.minimum(total, float(K))
    scale = 1.0 / (C * length)
    scale_ref[0] = jnp.broadcast_to(scale, (RB, CP))

    lg = log_r_ref[0]
    m = jnp.sum(jnp.abs(lg), axis=1, keepdims=True)
    part = jnp.sum((K - length) * m * scale).reshape(1, 1)

    first = jnp.logical_and(n == 0, i == 0)

    @pl.when(first)
    def _():
        offs_ref[...] = part

    @pl.when(jnp.logical_not(first))
    def _():
        offs_ref[...] = offs_ref[...] + part


def _tc_pack(points, logits):
    N = points.shape[0]
    ptsT = points.reshape(N, NQ, W, 3).transpose(0, 2, 3, 1)
    return pl.pallas_call(
        _pack_body,
        grid=(N, P // RB),
        in_specs=[
            pl.BlockSpec((1, RB, 3), lambda n, i: (n, i, 0)),
            pl.BlockSpec((1, W, 3, NQ), lambda n, i: (n, 0, 0, 0)),
            pl.BlockSpec((1, RB, C), lambda n, i: (n, i, 0)),
        ],
        out_specs=[
            pl.BlockSpec((1, RB, NQ), lambda n, i: (n, i, 0)),
            pl.BlockSpec((1, RB, CP), lambda n, i: (n, i, 0)),
            pl.BlockSpec((1, 1), lambda n, i: (0, 0)),
        ],
        out_shape=[
            jax.ShapeDtypeStruct((N, P, NQ), jnp.int32),
            jax.ShapeDtypeStruct((N, P, CP), jnp.float32),
            jax.ShapeDtypeStruct((1, 1), jnp.float32),
        ],
    )(points, ptsT, logits)


def _sc_select_gather_loss(logits_pad, packed_flat, scale2d):
    mesh = plsc.VectorSubcoreMesh(core_axis_name="c", subcore_axis_name="s")

    @functools.partial(
        pl.kernel,
        mesh=mesh,
        compiler_params=pltpu.CompilerParams(
            use_tc_tiling_on_sc=False, needs_layout_passes=False),
        out_type=jax.ShapeDtypeStruct((NTILES, CP), jnp.float32),
        scratch_types=[
            pltpu.VMEM((HALF, NQ), jnp.int32),
            pltpu.VMEM((HALF, CP), jnp.float32),
            pltpu.VMEM((HALF, CP), jnp.float32),
            pltpu.VMEM((HALF * K,), jnp.int32),
            pltpu.VMEM((HALF * K, CP), jnp.float32),
            pltpu.VMEM((32,), jnp.int32),
            pltpu.VMEM((32,), jnp.int32),
            pltpu.VMEM((CP,), jnp.float32),
            pltpu.SemaphoreType.DMA,
        ],
    )
    def sc_kernel(log_hbm, pk_hbm, scale_hbm, out_hbm,
                  pk_v, own_v, scale_v, idxb_v, rows_v, wv_buf, wb_buf,
                  acc_v, sem):
        wid = lax.axis_index("s") * 2 + lax.axis_index("c")
        iota16 = lax.broadcasted_iota(jnp.int32, (W,), 0)
        zeros16 = jnp.zeros((W,), jnp.int32)

        acc = jnp.zeros((CP,), jnp.float32)
        for half in range(PPT // HALF):
            base = wid * PPT + half * HALF
            pltpu.sync_copy(pk_hbm.at[pl.ds(base, HALF)], pk_v)
            pltpu.sync_copy(log_hbm.at[pl.ds(base, HALF)], own_v)
            pltpu.sync_copy(scale_hbm.at[pl.ds(base, HALF)], scale_v)
            cloud_off = jnp.where(base >= P, jnp.int32(P), jnp.int32(0))

            def row_body(r, carry):
                g_abs = base + r
                idxb_v[pl.ds(r * K, K)] = zeros16 + g_abs
                wv_buf[pl.ds(0, W)] = zeros16
                wv_buf[pl.ds(W, W)] = zeros16
                cnt = jnp.int32(0)
                for i in range(NQ // W):
                    w = pk_v[r, W * i:W * (i + 1)]
                    m = w != 0
                    mi = m.astype(jnp.int32)
                    pc = plsc.cumsum(mi)
                    keep = jnp.logical_and(m, (pc + cnt) <= K)
                    plsc.store_compressed(wv_buf.at[pl.ds(cnt, W)], w, mask=keep)
                    plsc.store_compressed(wb_buf.at[pl.ds(cnt, W)],
                                          iota16 + W * i, mask=keep)
                    cnt = cnt + jnp.sum(keep.astype(jnp.int32))
                cnt2 = jnp.int32(0)
                wv = wv_buf[pl.ds(0, W)]
                bv = wb_buf[pl.ds(0, W)]
                for l in range(W):
                    wl = jnp.sum(jnp.where(iota16 == l, wv, 0))
                    bl = jnp.sum(jnp.where(iota16 == l, bv, 0))
                    wspl = jnp.where((zeros16 + l) < cnt, zeros16 + wl, 0)
                    bits = jnp.bitwise_and(
                        jax.lax.shift_right_logical(wspl, iota16), 1)
                    m2 = bits == 1
                    pc2 = plsc.cumsum(bits)
                    keep2 = jnp.logical_and(m2, (pc2 + cnt2) <= K)
                    jv = bl * W + iota16 + cloud_off
                    plsc.store_compressed(
                        idxb_v.at[pl.ds(r * K + cnt2, W)], jv, mask=keep2)
                    cnt2 = cnt2 + jnp.sum(keep2.astype(jnp.int32))
                cur = idxb_v[pl.ds(r * K, K)]
                idxb_v[pl.ds(r * K, K)] = jnp.where(iota16 < cnt2, cur,
                                                    zeros16 + g_abs)
                return carry

            lax.fori_loop(0, HALF, row_body, 0)

            def gather_chunk(c, carry):
                pltpu.async_copy(
                    log_hbm.at[idxb_v.at[pl.ds(c * CHUNK, CHUNK)]],
                    rows_v.at[pl.ds(c * CHUNK, CHUNK)],
                    sem,
                ).wait()
                return carry

            lax.fori_loop(0, (HALF * K) // CHUNK, gather_chunk, 0)

            def point_body(p, a):
                own = own_v[p, :]
                sv = scale_v[p, :]
                for k in range(K):
                    nb = rows_v[p * K + k, :]
                    a = a + sv * jnp.abs(nb - own)
                return a

            acc = lax.fori_loop(0, HALF, point_body, acc)

        acc_v[...] = acc
        pltpu.sync_copy(acc_v, out_hbm.at[wid])

    return sc_kernel(logits_pad, packed_flat, scale2d)


def kernel(points, logits):
    N = points.shape[0]
    packed, scale, offs = _tc_pack(points, logits)
    logits_pad = jnp.pad(logits, ((0, 0), (0, 0), (0, CP - C)))
    logits_pad = logits_pad.reshape(N * P, CP)
    packed_flat = packed.reshape(N * P, NQ)
    scale2d = scale.reshape(N * P, CP)
    partials = _sc_select_gather_loss(logits_pad, packed_flat, scale2d)
    return (jnp.sum(partials) + offs[0, 0]) / (N * P)


# speedup vs baseline: 2.5634x; 1.1386x over previous
import functools
import jax
import jax.numpy as jnp
from jax import lax
from jax.experimental import pallas as pl
from jax.experimental.pallas import tpu as pltpu
from jax.experimental.pallas import tpu_sc as plsc

P = 4096
K = 16
C = 13
CP = 16
RADIUS2 = 0.01
RB = 256
W = 16
NQ = P // W
NTILES = 32
PPT = 256
HALF = 128
CHUNK = 128


def _lane_cumsum(x):
    n = x.shape[-1]
    lane = jax.lax.broadcasted_iota(jnp.int32, x.shape, len(x.shape) - 1)
    s = 1
    while s < n:
        shifted = pltpu.roll(x, s, axis=len(x.shape) - 1)
        x = x + jnp.where(lane >= s, shifted, 0.0)
        s *= 2
    return x


def _pack_body(pts_r_ref, ptsT_ref, log_r_ref, packed_ref, scale_ref, offs_ref):
    n = pl.program_id(0)
    i = pl.program_id(1)

    x = pts_r_ref[0]
    words = jnp.zeros((RB, NQ), jnp.int32)
    cnt = jnp.zeros((RB, NQ), jnp.float32)
    for b in range(W):
        xb = ptsT_ref[0, b]
        d2 = jnp.zeros((RB, NQ), jnp.float32)
        for d in range(3):
            t = x[:, d:d + 1] - xb[d:d + 1, :]
            d2 = d2 + t * t
        wb = d2 < RADIUS2
        words = words + jnp.where(wb, jnp.int32(1 << b), jnp.int32(0))
        cnt = cnt + wb.astype(jnp.float32)
    bit_pfx = _lane_cumsum(cnt) - cnt
    nzw = (cnt > 0.0).astype(jnp.float32)
    nz_pfx = _lane_cumsum(nzw) - nzw
    bp = jnp.minimum(bit_pfx, 17.0).astype(jnp.int32)
    npx = jnp.minimum(nz_pfx, 17.0).astype(jnp.int32)
    packed_ref[0] = words | (bp << 16) | (npx << 21)

    total = jnp.sum(cnt, axis=1, keepdims=True)
    length = jnp.minimum(total, float(K))
    scale = 1.0 / (C * length)
    scale_ref[0] = jnp.broadcast_to(scale, (RB, CP))

    lg = log_r_ref[0]
    m = jnp.sum(jnp.abs(lg), axis=1, keepdims=True)
    part = jnp.sum((K - length) * m * scale).reshape(1, 1)

    first = jnp.logical_and(n == 0, i == 0)

    @pl.when(first)
    def _():
        offs_ref[...] = part

    @pl.when(jnp.logical_not(first))
    def _():
        offs_ref[...] = offs_ref[...] + part


def _tc_pack(points, logits):
    N = points.shape[0]
    ptsT = points.reshape(N, NQ, W, 3).transpose(0, 2, 3, 1)
    return pl.pallas_call(
        _pack_body,
        grid=(N, P // RB),
        in_specs=[
            pl.BlockSpec((1, RB, 3), lambda n, i: (n, i, 0)),
            pl.BlockSpec((1, W, 3, NQ), lambda n, i: (n, 0, 0, 0)),
            pl.BlockSpec((1, RB, C), lambda n, i: (n, i, 0)),
        ],
        out_specs=[
            pl.BlockSpec((1, RB, NQ), lambda n, i: (n, i, 0)),
            pl.BlockSpec((1, RB, CP), lambda n, i: (n, i, 0)),
            pl.BlockSpec((1, 1), lambda n, i: (0, 0)),
        ],
        out_shape=[
            jax.ShapeDtypeStruct((N, P, NQ), jnp.int32),
            jax.ShapeDtypeStruct((N, P, CP), jnp.float32),
            jax.ShapeDtypeStruct((1, 1), jnp.float32),
        ],
    )(points, ptsT, logits)


def _sc_select_gather_loss(logits_pad, packed_flat, scale2d):
    mesh = plsc.VectorSubcoreMesh(core_axis_name="c", subcore_axis_name="s")

    @functools.partial(
        pl.kernel,
        mesh=mesh,
        compiler_params=pltpu.CompilerParams(
            use_tc_tiling_on_sc=False, needs_layout_passes=False),
        out_type=jax.ShapeDtypeStruct((NTILES, CP), jnp.float32),
        scratch_types=[
            pltpu.VMEM((HALF, NQ), jnp.int32),
            pltpu.VMEM((HALF, CP), jnp.float32),
            pltpu.VMEM((HALF, CP), jnp.float32),
            pltpu.VMEM((HALF * K + W,), jnp.int32),
            pltpu.VMEM((HALF * K, CP), jnp.float32),
            pltpu.VMEM((32,), jnp.int32),
            pltpu.VMEM((32,), jnp.int32),
            pltpu.VMEM((CP,), jnp.float32),
            pltpu.SemaphoreType.DMA,
        ],
    )
    def sc_kernel(log_hbm, pk_hbm, scale_hbm, out_hbm,
                  pk_v, own_v, scale_v, idxb_v, rows_v, wv_buf, wb_buf,
                  acc_v, sem):
        wid = lax.axis_index("s") * 2 + lax.axis_index("c")
        iota16 = lax.broadcasted_iota(jnp.int32, (W,), 0)
        zeros16 = jnp.zeros((W,), jnp.int32)
        below_mask = (jnp.int32(1) << iota16) - 1
        gdims = lax.GatherDimensionNumbers(
            offset_dims=(), collapsed_slice_dims=(0,), start_index_map=(0,))

        def splat(vec, l):
            return lax.gather(vec, (zeros16 + l).reshape(W, 1), gdims, (1,),
                              mode=lax.GatherScatterMode.PROMISE_IN_BOUNDS)

        wv_buf[pl.ds(0, W)] = zeros16
        wv_buf[pl.ds(W, W)] = zeros16
        acc = jnp.zeros((CP,), jnp.float32)
        for half in range(PPT // HALF):
            base = wid * PPT + half * HALF
            pltpu.sync_copy(pk_hbm.at[pl.ds(base, HALF)], pk_v)
            pltpu.sync_copy(log_hbm.at[pl.ds(base, HALF)], own_v)
            pltpu.sync_copy(scale_hbm.at[pl.ds(base, HALF)], scale_v)
            cloud_off = jnp.where(base >= P, jnp.int32(P), jnp.int32(0))

            def prefill(r, carry):
                idxb_v[pl.ds(r * K, K)] = zeros16 + (base + r)
                return carry

            lax.fori_loop(0, HALF, prefill, 0)

            def row_body(r, carry):
                tag = half * HALF + r
                for i in range(NQ // W):
                    cw = pk_v[r, W * i:W * (i + 1)]
                    w = cw & 0xFFFF
                    bp = (cw >> 16) & 0x1F
                    npx = (cw >> 21) & 0x1F
                    m = jnp.logical_and(w != 0, bp < K)
                    pos = jnp.where(m, npx, jnp.int32(31))
                    val = (cw & 0x1FFFFF) | (tag << 21)
                    plsc.store_scatter(wv_buf, [pos], val)
                    plsc.store_scatter(wb_buf, [pos], iota16 + W * i)
                wv = wv_buf[pl.ds(0, W)]
                bv = wb_buf[pl.ds(0, W)]
                for l in range(W):
                    cwl = splat(wv, l)
                    bl = splat(bv, l)
                    valid = ((cwl >> 21) & 0xFF) == tag
                    wl = cwl & 0xFFFF
                    bpl = (cwl >> 16) & 0x1F
                    bits = jnp.bitwise_and(
                        jax.lax.shift_right_logical(wl, iota16), 1)
                    x = wl & below_mask
                    x = x - ((x >> 1) & 0x5555)
                    x = (x & 0x3333) + ((x >> 2) & 0x3333)
                    x = (x + (x >> 4)) & 0x0F0F
                    pc = (x + (x >> 8)) & 0x1F
                    slot = bpl + pc
                    keep = jnp.logical_and(
                        jnp.logical_and(bits == 1, slot < K), valid)
                    addr = jnp.where(keep, r * K + slot, jnp.int32(HALF * K))
                    jv = bl * W + iota16 + cloud_off
                    plsc.store_scatter(idxb_v, [addr], jv)
                return carry

            lax.fori_loop(0, HALF, row_body, 0)

            def gather_chunk(c, carry):
                pltpu.async_copy(
                    log_hbm.at[idxb_v.at[pl.ds(c * CHUNK, CHUNK)]],
                    rows_v.at[pl.ds(c * CHUNK, CHUNK)],
                    sem,
                ).wait()
                return carry

            lax.fori_loop(0, (HALF * K) // CHUNK, gather_chunk, 0)

            def point_body(p, a):
                own = own_v[p, :]
                sv = scale_v[p, :]
                for k in range(K):
                    nb = rows_v[p * K + k, :]
                    a = a + sv * jnp.abs(nb - own)
                return a

            acc = lax.fori_loop(0, HALF, point_body, acc)

        acc_v[...] = acc
        pltpu.sync_copy(acc_v, out_hbm.at[wid])

    return sc_kernel(logits_pad, packed_flat, scale2d)


def kernel(points, logits):
    N = points.shape[0]
    packed, scale, offs = _tc_pack(points, logits)
    logits_pad = jnp.pad(logits, ((0, 0), (0, 0), (0, CP - C)))
    logits_pad = logits_pad.reshape(N * P, CP)
    packed_flat = packed.reshape(N * P, NQ)
    scale2d = scale.reshape(N * P, CP)
    partials = _sc_select_gather_loss(logits_pad, packed_flat, scale2d)
    return (jnp.sum(partials) + offs[0, 0]) / (N * P)


# speedup vs baseline: 3.1857x; 1.2427x over previous
import functools
import jax
import jax.numpy as jnp
from jax import lax
from jax.experimental import pallas as pl
from jax.experimental.pallas import tpu as pltpu
from jax.experimental.pallas import tpu_sc as plsc

P = 4096
K = 16
C = 13
CP = 16
RADIUS2 = 0.01
RB = 256
W = 16
NQ = P // W
NTILES = 32
PPT = 128
HALF = 128
CHUNK = 128


def _lane_cumsum(x):
    n = x.shape[-1]
    lane = jax.lax.broadcasted_iota(jnp.int32, x.shape, len(x.shape) - 1)
    s = 1
    while s < n:
        shifted = pltpu.roll(x, s, axis=len(x.shape) - 1)
        x = x + jnp.where(lane >= s, shifted, 0.0)
        s *= 2
    return x


def _pack_body(pts_r_ref, ptsT_ref, log_r_ref, packed_ref, scale_ref, offs_ref):
    n = pl.program_id(0)
    i = pl.program_id(1)

    x = pts_r_ref[0]
    words = jnp.zeros((RB, NQ), jnp.int32)
    cnt = jnp.zeros((RB, NQ), jnp.float32)
    for b in range(W):
        xb = ptsT_ref[0, b]
        d2 = jnp.zeros((RB, NQ), jnp.float32)
        for d in range(3):
            t = x[:, d:d + 1] - xb[d:d + 1, :]
            d2 = d2 + t * t
        wb = d2 < RADIUS2
        words = words + jnp.where(wb, jnp.int32(1 << b), jnp.int32(0))
        cnt = cnt + wb.astype(jnp.float32)
    bit_pfx = _lane_cumsum(cnt) - cnt
    nzw = (cnt > 0.0).astype(jnp.float32)
    nz_pfx = _lane_cumsum(nzw) - nzw
    bp = jnp.minimum(bit_pfx, 17.0).astype(jnp.int32)
    npx = jnp.minimum(nz_pfx, 17.0).astype(jnp.int32)
    packed_ref[0] = words | (bp << 16) | (npx << 21)

    total = jnp.sum(cnt, axis=1, keepdims=True)
    length = jnp.minimum(total, float(K))
    scale = 1.0 / (C * length)
    scale_ref[0] = jnp.broadcast_to(scale, (RB, CP))

    lg = log_r_ref[0]
    m = jnp.sum(jnp.abs(lg), axis=1, keepdims=True)
    part = jnp.sum((K - length) * m * scale).reshape(1, 1)

    first = jnp.logical_and(n == 0, i == 0)

    @pl.when(first)
    def _():
        offs_ref[...] = part

    @pl.when(jnp.logical_not(first))
    def _():
        offs_ref[...] = offs_ref[...] + part


def _tc_pack(points, logits):
    N = points.shape[0]
    ptsT = points.reshape(N, NQ, W, 3).transpose(0, 2, 3, 1)
    return pl.pallas_call(
        _pack_body,
        grid=(N, P // RB),
        in_specs=[
            pl.BlockSpec((1, RB, 3), lambda n, i: (n, i, 0)),
            pl.BlockSpec((1, W, 3, NQ), lambda n, i: (n, 0, 0, 0)),
            pl.BlockSpec((1, RB, C), lambda n, i: (n, i, 0)),
        ],
        out_specs=[
            pl.BlockSpec((1, RB, NQ), lambda n, i: (n, i, 0)),
            pl.BlockSpec((1, RB, CP), lambda n, i: (n, i, 0)),
            pl.BlockSpec((1, 1), lambda n, i: (0, 0)),
        ],
        out_shape=[
            jax.ShapeDtypeStruct((N, P, NQ), jnp.int32),
            jax.ShapeDtypeStruct((N, P, CP), jnp.float32),
            jax.ShapeDtypeStruct((1, 1), jnp.float32),
        ],
    )(points, ptsT, logits)


def _sc_select_gather_loss(logits_pad, packed_flat, scale2d):
    mesh = plsc.VectorSubcoreMesh(core_axis_name="c", subcore_axis_name="s")

    @functools.partial(
        pl.kernel,
        mesh=mesh,
        compiler_params=pltpu.CompilerParams(
            use_tc_tiling_on_sc=False, needs_layout_passes=False),
        out_type=jax.ShapeDtypeStruct((NTILES, CP), jnp.float32),
        scratch_types=[
            pltpu.VMEM((HALF, NQ), jnp.int32),
            pltpu.VMEM((HALF, CP), jnp.float32),
            pltpu.VMEM((HALF, CP), jnp.float32),
            pltpu.VMEM((HALF * K + W,), jnp.int32),
            pltpu.VMEM((HALF * K, CP), jnp.float32),
            pltpu.VMEM((32,), jnp.int32),
            pltpu.VMEM((32,), jnp.int32),
            pltpu.VMEM((CP,), jnp.float32),
            pltpu.SemaphoreType.DMA,
        ],
    )
    def sc_kernel(log_hbm, pk_hbm, scale_hbm, out_hbm,
                  pk_v, own_v, scale_v, idxb_v, rows_v, wv_buf, wb_buf,
                  acc_v, sem):
        wid = lax.axis_index("s") * 2 + lax.axis_index("c")
        iota16 = lax.broadcasted_iota(jnp.int32, (W,), 0)
        zeros16 = jnp.zeros((W,), jnp.int32)
        below_mask = (jnp.int32(1) << iota16) - 1
        gdims = lax.GatherDimensionNumbers(
            offset_dims=(), collapsed_slice_dims=(0,), start_index_map=(0,))

        def splat(vec, l):
            return lax.gather(vec, (zeros16 + l).reshape(W, 1), gdims, (1,),
                              mode=lax.GatherScatterMode.PROMISE_IN_BOUNDS)

        wv_buf[pl.ds(0, W)] = zeros16
        wv_buf[pl.ds(W, W)] = zeros16
        acc = jnp.zeros((CP,), jnp.float32)
        for half in range(PPT // HALF):
            base = wid * PPT + half * HALF
            pltpu.sync_copy(pk_hbm.at[pl.ds(base, HALF)], pk_v)
            pltpu.sync_copy(log_hbm.at[pl.ds(base, HALF)], own_v)
            pltpu.sync_copy(scale_hbm.at[pl.ds(base, HALF)], scale_v)
            cloud_off = jnp.where(base >= P, jnp.int32(P), jnp.int32(0))

            def prefill(r, carry):
                idxb_v[pl.ds(r * K, K)] = zeros16 + (base + r)
                return carry

            lax.fori_loop(0, HALF, prefill, 0)

            def row_body(r, carry):
                tag = half * HALF + r
                for i in range(NQ // W):
                    cw = pk_v[r, W * i:W * (i + 1)]
                    w = cw & 0xFFFF
                    bp = (cw >> 16) & 0x1F
                    npx = (cw >> 21) & 0x1F
                    m = jnp.logical_and(w != 0, bp < K)
                    pos = jnp.where(m, npx, jnp.int32(31))
                    val = (cw & 0x1FFFFF) | (tag << 21)
                    plsc.store_scatter(wv_buf, [pos], val)
                    plsc.store_scatter(wb_buf, [pos], iota16 + W * i)
                wv = wv_buf[pl.ds(0, W)]
                bv = wb_buf[pl.ds(0, W)]
                for l in range(W):
                    cwl = splat(wv, l)
                    bl = splat(bv, l)
                    valid = ((cwl >> 21) & 0xFF) == tag
                    wl = cwl & 0xFFFF
                    bpl = (cwl >> 16) & 0x1F
                    bits = jnp.bitwise_and(
                        jax.lax.shift_right_logical(wl, iota16), 1)
                    x = wl & below_mask
                    x = x - ((x >> 1) & 0x5555)
                    x = (x & 0x3333) + ((x >> 2) & 0x3333)
                    x = (x + (x >> 4)) & 0x0F0F
                    pc = (x + (x >> 8)) & 0x1F
                    slot = bpl + pc
                    keep = jnp.logical_and(
                        jnp.logical_and(bits == 1, slot < K), valid)
                    addr = jnp.where(keep, r * K + slot, jnp.int32(HALF * K))
                    jv = bl * W + iota16 + cloud_off
                    plsc.store_scatter(idxb_v, [addr], jv)
                return carry

            lax.fori_loop(0, HALF, row_body, 0)

            copies = [
                pltpu.async_copy(
                    log_hbm.at[idxb_v.at[pl.ds(c * CHUNK, CHUNK)]],
                    rows_v.at[pl.ds(c * CHUNK, CHUNK)],
                    sem,
                )
                for c in range((HALF * K) // CHUNK)
            ]
            for cp in copies:
                cp.wait()

            def point_body(p, a):
                own = own_v[p, :]
                sv = scale_v[p, :]
                for k in range(K):
                    nb = rows_v[p * K + k, :]
                    a = a + sv * jnp.abs(nb - own)
                return a

            acc = lax.fori_loop(0, HALF, point_body, acc)

        acc_v[...] = acc
        pltpu.sync_copy(acc_v, out_hbm.at[wid])

    return sc_kernel(logits_pad, packed_flat, scale2d)


def kernel(points, logits):
    N = points.shape[0]
    total = jnp.float32(0.0)
    for n in range(N):
        packed, scale, offs = _tc_pack(points[n:n + 1], logits[n:n + 1])
        lp = jnp.pad(logits[n], ((0, 0), (0, CP - C)))
        partials = _sc_select_gather_loss(
            lp, packed.reshape(P, NQ), scale.reshape(P, CP))
        total = total + jnp.sum(partials) + offs[0, 0]
    return total / (N * P)
